# baseline (speedup 1.0000x reference)
import functools

import jax
import jax.numpy as jnp
from jax import lax
from jax.experimental import pallas as pl
from jax.experimental.pallas import tpu as pltpu
from jax.experimental.pallas import tpu_sc as plsc


def _sc_gather(table, idx_flat, E, CH):
    info = plsc.get_sparse_core_info()
    NC, NS = info.num_cores, info.num_subcores
    NW = NC * NS
    n_rows = idx_flat.shape[0]
    b_per_w = n_rows // NW
    ch_per_w = b_per_w // CH

    mesh = plsc.VectorSubcoreMesh(core_axis_name="c", subcore_axis_name="s")

    @functools.partial(
        pl.kernel,
        out_type=jax.ShapeDtypeStruct((n_rows, E), jnp.float32),
        mesh=mesh,
        scratch_types=[
            pltpu.VMEM((b_per_w,), jnp.int32),
            pltpu.VMEM((b_per_w, E), jnp.float32),
            pltpu.SemaphoreType.DMA,
        ],
        compiler_params=pltpu.CompilerParams(use_tc_tiling_on_sc=False),
    )
    def gather_k(table_hbm, idx_hbm, out_hbm, idx_v, rows_v, sem):
        wid = lax.axis_index("s") * NC + lax.axis_index("c")
        base = wid * b_per_w
        pltpu.sync_copy(idx_hbm.at[pl.ds(base, b_per_w)], idx_v)

        offs = list(range(0, b_per_w, CH))
        cps = [
            pltpu.async_copy(
                table_hbm.at[idx_v.at[pl.ds(o, min(CH, b_per_w - o))]],
                rows_v.at[pl.ds(o, min(CH, b_per_w - o))],
                sem,
            )
            for o in offs
        ]
        for cp in cps:
            cp.wait()
        pltpu.sync_copy(rows_v, out_hbm.at[pl.ds(base, b_per_w)])

    return gather_k(table, idx_flat)


def _lstm_body(emb_ref, wih_ref, whh_ref, b_ref, out_ref):
    BB, T, E = emb_ref.shape
    H = out_ref.shape[1]
    wih = wih_ref[...]
    whh = whh_ref[...]
    b = b_ref[...]

    def step(t, carry):
        h, c = carry
        x_t = emb_ref[:, t, :]
        gates = (
            jnp.dot(x_t, wih, preferred_element_type=jnp.float32)
            + jnp.dot(h, whh, preferred_element_type=jnp.float32)
            + b
        )
        i = jax.nn.sigmoid(gates[:, 0 * H:1 * H])
        f = jax.nn.sigmoid(gates[:, 1 * H:2 * H])
        g = jnp.tanh(gates[:, 2 * H:3 * H])
        o = jax.nn.sigmoid(gates[:, 3 * H:4 * H])
        c_new = f * c + i * g
        h_new = o * jnp.tanh(c_new)
        return (h_new, c_new)

    h0 = jnp.zeros((BB, H), jnp.float32)
    c0 = jnp.zeros((BB, H), jnp.float32)
    h_last, _ = lax.fori_loop(0, T, step, (h0, c0))
    out_ref[...] = h_last


def _lstm(emb_seq, W_ihT, W_hhT, bias2d):
    B, T, E = emb_seq.shape
    H4 = W_ihT.shape[1]
    H = H4 // 4
    BB = B
    return pl.pallas_call(
        _lstm_body,
        grid=(B // BB,),
        in_specs=[
            pl.BlockSpec((BB, T, E), lambda i: (i, 0, 0)),
            pl.BlockSpec((E, H4), lambda i: (0, 0)),
            pl.BlockSpec((H, H4), lambda i: (0, 0)),
            pl.BlockSpec((1, H4), lambda i: (0, 0)),
        ],
        out_specs=pl.BlockSpec((BB, H), lambda i: (i, 0)),
        out_shape=jax.ShapeDtypeStruct((B, H), jnp.float32),
    )(emb_seq, W_ihT, W_hhT, bias2d)


_VB = 4096
_NQ = 4


_VB = 4096
_KF = 5
_FW = _VB * _KF
_NF = 5
_RB = 64


def _fc_body(h_ref, w_hbm, b_ref, out_ref, w_v, rowbuf, tailbuf, nsems, sm_sem, st_sem, w_sem):
    V = out_ref.shape[1]
    G = out_ref.shape[0] // _RB
    last_f = _NF - 1
    base4 = last_f * _FW
    main4 = (V - base4) // 128 * 128
    rem4 = V - base4 - main4
    tf_off = base4 + (_KF - 1) * _VB
    roff = (base4 + main4) - tf_off

    g = pl.program_id(0)
    f = pl.program_id(1)
    idx = g * _NF + f
    par = lax.rem(idx, 2)

    @pl.when(idx == 0)
    def _():
        pltpu.make_async_copy(w_hbm, w_v, w_sem).start()
        pltpu.make_async_copy(w_hbm, w_v, w_sem).wait()

    @pl.when(jnp.logical_and(idx >= 2, f != 1))
    def _():
        pltpu.make_async_copy(
            rowbuf.at[par],
            out_ref.at[pl.ds(0, _RB), pl.ds(0, _FW)],
            nsems.at[par],
        ).wait()

    @pl.when(jnp.logical_and(idx >= 2, f == 1))
    def _():
        pltpu.make_async_copy(
            rowbuf.at[par, :, pl.ds(0, main4)],
            out_ref.at[pl.ds(0, _RB), pl.ds(0, main4)],
            sm_sem.at[par],
        ).wait()
        pltpu.make_async_copy(
            tailbuf.at[par],
            out_ref.at[pl.ds(0, _RB), pl.ds(base4 + main4, rem4)],
            st_sem.at[par],
        ).wait()

    hv = h_ref[...]
    for k in range(_KF):
        wk = w_v[pl.ds((f * _KF + k) * _VB, _VB), :]
        vals = (
            lax.dot_general(
                hv, wk,
                dimension_numbers=(((1,), (1,)), ((), ())),
                preferred_element_type=jnp.float32,
            )
            + b_ref[:, k * _VB:(k + 1) * _VB]
        )
        rowbuf[par, :, k * _VB:(k + 1) * _VB] = vals
        if k == _KF - 1:
            tailbuf[par] = vals[:, roff:roff + rem4]

    @pl.when(f < last_f)
    def _():
        pltpu.make_async_copy(
            rowbuf.at[par],
            out_ref.at[pl.ds(g * _RB, _RB), pl.ds(f * _FW, _FW)],
            nsems.at[par],
        ).start()

    @pl.when(f == last_f)
    def _():
        pltpu.make_async_copy(
            rowbuf.at[par, :, pl.ds(0, main4)],
            out_ref.at[pl.ds(g * _RB, _RB), pl.ds(base4, main4)],
            sm_sem.at[par],
        ).start()
        pltpu.make_async_copy(
            tailbuf.at[par],
            out_ref.at[pl.ds(g * _RB, _RB), pl.ds(base4 + main4, rem4)],
            st_sem.at[par],
        ).start()

    @pl.when(jnp.logical_and(g == G - 1, f == last_f))
    def _():
        pltpu.make_async_copy(
            rowbuf.at[1 - par],
            out_ref.at[pl.ds(0, _RB), pl.ds(0, _FW)],
            nsems.at[1 - par],
        ).wait()
        pltpu.make_async_copy(
            rowbuf.at[par, :, pl.ds(0, main4)],
            out_ref.at[pl.ds(0, _RB), pl.ds(0, main4)],
            sm_sem.at[par],
        ).wait()
        pltpu.make_async_copy(
            tailbuf.at[par],
            out_ref.at[pl.ds(0, _RB), pl.ds(base4 + main4, rem4)],
            st_sem.at[par],
        ).wait()


def _fc(h, W_pad, b_pad2d, V):
    B, H = h.shape
    Vp = W_pad.shape[0]
    rem = (V - (_NF - 1) * _FW) % 128
    return pl.pallas_call(
        _fc_body,
        grid=(B // _RB, _NF),
        in_specs=[
            pl.BlockSpec((_RB, H), lambda g, f: (g, 0)),
            pl.BlockSpec(memory_space=pl.ANY),
            pl.BlockSpec((1, _FW), lambda g, f: (0, f)),
        ],
        out_specs=pl.BlockSpec(memory_space=pl.ANY),
        out_shape=jax.ShapeDtypeStruct((B, V), jnp.float32),
        scratch_shapes=[
            pltpu.VMEM((Vp, H), jnp.bfloat16),
            pltpu.VMEM((2, _RB, _FW), jnp.float32),
            pltpu.VMEM((2, _RB, rem), jnp.float32),
            pltpu.SemaphoreType.DMA((2,)),
            pltpu.SemaphoreType.DMA((2,)),
            pltpu.SemaphoreType.DMA((2,)),
            pltpu.SemaphoreType.DMA,
        ],
    )(h, W_pad, b_pad2d)


def kernel(x, emb, W_ih, W_hh, b_ih, b_hh, W_fc, b_fc):
    B, T = x.shape
    V, E = emb.shape
    H = W_hh.shape[1]


    CH = 128
    idx_flat = x.astype(jnp.int32).reshape(-1)

    embedded = _sc_gather(emb, idx_flat, E, CH).reshape(B, T, E)

    h_last = _lstm(
        embedded,
        W_ih.T,
        W_hh.T,
        (b_ih + b_hh).reshape(1, 4 * H),
    )

    Vp = _NF * _FW
    W_pad = jnp.pad(W_fc.astype(jnp.bfloat16), ((0, Vp - V), (0, 0)))
    b_pad = jnp.pad(b_fc, (0, Vp - V)).reshape(1, Vp)
    return _fc(h_last.astype(jnp.bfloat16), W_pad, b_pad, V)


# speedup vs baseline: 1.0530x; 1.0530x over previous
import functools

import jax
import jax.numpy as jnp
from jax import lax
from jax.experimental import pallas as pl
from jax.experimental.pallas import tpu as pltpu
from jax.experimental.pallas import tpu_sc as plsc


def _sc_gather(table, idx_flat, E, CH):
    info = plsc.get_sparse_core_info()
    NC, NS = info.num_cores, info.num_subcores
    NW = NC * NS
    n_rows = idx_flat.shape[0]
    b_per_w = n_rows // NW
    ch_per_w = b_per_w // CH

    mesh = plsc.VectorSubcoreMesh(core_axis_name="c", subcore_axis_name="s")

    @functools.partial(
        pl.kernel,
        out_type=jax.ShapeDtypeStruct((n_rows, E), jnp.float32),
        mesh=mesh,
        scratch_types=[
            pltpu.VMEM((b_per_w,), jnp.int32),
            pltpu.VMEM((b_per_w, E), jnp.float32),
            pltpu.SemaphoreType.DMA,
        ],
        compiler_params=pltpu.CompilerParams(use_tc_tiling_on_sc=False),
    )
    def gather_k(table_hbm, idx_hbm, out_hbm, idx_v, rows_v, sem):
        wid = lax.axis_index("s") * NC + lax.axis_index("c")
        base = wid * b_per_w
        pltpu.sync_copy(idx_hbm.at[pl.ds(base, b_per_w)], idx_v)

        offs = list(range(0, b_per_w, CH))
        cps = [
            pltpu.async_copy(
                table_hbm.at[idx_v.at[pl.ds(o, min(CH, b_per_w - o))]],
                rows_v.at[pl.ds(o, min(CH, b_per_w - o))],
                sem,
            )
            for o in offs
        ]
        for cp in cps:
            cp.wait()
        pltpu.sync_copy(rows_v, out_hbm.at[pl.ds(base, b_per_w)])

    return gather_k(table, idx_flat)


def _lstm_body(emb_ref, wih_ref, whh_ref, b_ref, out_ref):
    BB, T, E = emb_ref.shape
    H = out_ref.shape[1]
    wih = wih_ref[...]
    whh = whh_ref[...]
    b = b_ref[...]

    def step(t, carry):
        h, c = carry
        x_t = emb_ref[:, t, :]
        gates = (
            jnp.dot(x_t, wih, preferred_element_type=jnp.float32)
            + jnp.dot(h, whh, preferred_element_type=jnp.float32)
            + b
        )
        i = jax.nn.sigmoid(gates[:, 0 * H:1 * H])
        f = jax.nn.sigmoid(gates[:, 1 * H:2 * H])
        g = jnp.tanh(gates[:, 2 * H:3 * H])
        o = jax.nn.sigmoid(gates[:, 3 * H:4 * H])
        c_new = f * c + i * g
        h_new = o * jnp.tanh(c_new)
        return (h_new, c_new)

    h0 = jnp.zeros((BB, H), jnp.float32)
    c0 = jnp.zeros((BB, H), jnp.float32)
    h_last, _ = lax.fori_loop(0, T, step, (h0, c0))
    out_ref[...] = h_last


def _lstm(emb_seq, W_ihT, W_hhT, bias2d):
    B, T, E = emb_seq.shape
    H4 = W_ihT.shape[1]
    H = H4 // 4
    BB = B
    return pl.pallas_call(
        _lstm_body,
        grid=(B // BB,),
        in_specs=[
            pl.BlockSpec((BB, T, E), lambda i: (i, 0, 0)),
            pl.BlockSpec((E, H4), lambda i: (0, 0)),
            pl.BlockSpec((H, H4), lambda i: (0, 0)),
            pl.BlockSpec((1, H4), lambda i: (0, 0)),
        ],
        out_specs=pl.BlockSpec((BB, H), lambda i: (i, 0)),
        out_shape=jax.ShapeDtypeStruct((B, H), jnp.float32),
    )(emb_seq, W_ihT, W_hhT, bias2d)


_VB = 4096
_NQ = 4


def _fc_body(h_ref, w_ref, b_ref, out_ref):
    out_ref[...] = (
        lax.dot_general(
            h_ref[...], w_ref[...],
            dimension_numbers=(((1,), (1,)), ((), ())),
            preferred_element_type=jnp.float32,
        )
        + b_ref[...]
    )


def _fc(h, W_fc, b_fc2d):
    B, H = h.shape
    V = W_fc.shape[0]
    VB = 4096
    nv = pl.cdiv(V, VB)
    return pl.pallas_call(
        _fc_body,
        grid=(nv,),
        in_specs=[
            pl.BlockSpec((B, H), lambda i: (0, 0)),
            pl.BlockSpec((VB, H), lambda i: (i, 0)),
            pl.BlockSpec((1, VB), lambda i: (0, i)),
        ],
        out_specs=pl.BlockSpec((B, VB), lambda i: (0, i)),
        out_shape=jax.ShapeDtypeStruct((B, V), jnp.float32),
    )(h, W_fc, b_fc2d)


def kernel(x, emb, W_ih, W_hh, b_ih, b_hh, W_fc, b_fc):
    B, T = x.shape
    V, E = emb.shape
    H = W_hh.shape[1]


    CH = 128
    idx_flat = x.astype(jnp.int32).reshape(-1)

    embedded = _sc_gather(emb, idx_flat, E, CH).reshape(B, T, E)

    h_last = _lstm(
        embedded,
        W_ih.T,
        W_hh.T,
        (b_ih + b_hh).reshape(1, 4 * H),
    )

    return _fc(h_last.astype(jnp.bfloat16), W_fc.astype(jnp.bfloat16),
               b_fc.reshape(1, V))


# speedup vs baseline: 1.0562x; 1.0031x over previous
import functools

import jax
import jax.numpy as jnp
from jax import lax
from jax.experimental import pallas as pl
from jax.experimental.pallas import tpu as pltpu
from jax.experimental.pallas import tpu_sc as plsc


def _sc_gather(table, idx_flat, E, CH):
    info = plsc.get_sparse_core_info()
    NC, NS = info.num_cores, info.num_subcores
    NW = NC * NS
    n_rows = idx_flat.shape[0]
    b_per_w = n_rows // NW

    mesh = plsc.VectorSubcoreMesh(core_axis_name="c", subcore_axis_name="s")

    @functools.partial(
        pl.kernel,
        out_type=jax.ShapeDtypeStruct((n_rows, E), jnp.float32),
        mesh=mesh,
        scratch_types=[
            pltpu.VMEM((b_per_w,), jnp.int32),
            pltpu.VMEM((b_per_w, E), jnp.float32),
            pltpu.SemaphoreType.DMA,
        ],
        compiler_params=pltpu.CompilerParams(use_tc_tiling_on_sc=False),
    )
    def gather_k(table_hbm, idx_hbm, out_hbm, idx_v, rows_v, sem):
        wid = lax.axis_index("s") * NC + lax.axis_index("c")
        base = wid * b_per_w
        pltpu.sync_copy(idx_hbm.at[pl.ds(base, b_per_w)], idx_v)

        offs = list(range(0, b_per_w, CH))
        cps = [
            pltpu.async_copy(
                table_hbm.at[idx_v.at[pl.ds(o, min(CH, b_per_w - o))]],
                rows_v.at[pl.ds(o, min(CH, b_per_w - o))],
                sem,
            )
            for o in offs
        ]
        for cp in cps:
            cp.wait()
        pltpu.sync_copy(rows_v, out_hbm.at[pl.ds(base, b_per_w)])

    return gather_k(table, idx_flat)


def _lstm_body(emb_ref, wih_ref, whh_ref, b_ref, out_ref):
    BB, T, E = emb_ref.shape
    H = out_ref.shape[1]
    wih = wih_ref[...]
    whh = whh_ref[...]
    b = b_ref[...]

    def step(t, carry):
        h, c = carry
        x_t = emb_ref[:, t, :]
        gates = (
            jnp.dot(x_t, wih, preferred_element_type=jnp.float32)
            + jnp.dot(h, whh, preferred_element_type=jnp.float32)
            + b
        )
        i = jax.nn.sigmoid(gates[:, 0 * H:1 * H])
        f = jax.nn.sigmoid(gates[:, 1 * H:2 * H])
        g = jnp.tanh(gates[:, 2 * H:3 * H])
        o = jax.nn.sigmoid(gates[:, 3 * H:4 * H])
        c_new = f * c + i * g
        h_new = o * jnp.tanh(c_new)
        return (h_new, c_new)

    h0 = jnp.zeros((BB, H), jnp.float32)
    c0 = jnp.zeros((BB, H), jnp.float32)
    h_last, _ = lax.fori_loop(0, T, step, (h0, c0))
    out_ref[...] = h_last


def _lstm(emb_seq, W_ihT, W_hhT, bias2d):
    B, T, E = emb_seq.shape
    H4 = W_ihT.shape[1]
    H = H4 // 4
    BB = B
    return pl.pallas_call(
        _lstm_body,
        grid=(B // BB,),
        in_specs=[
            pl.BlockSpec((BB, T, E), lambda i: (i, 0, 0)),
            pl.BlockSpec((E, H4), lambda i: (0, 0)),
            pl.BlockSpec((H, H4), lambda i: (0, 0)),
            pl.BlockSpec((1, H4), lambda i: (0, 0)),
        ],
        out_specs=pl.BlockSpec((BB, H), lambda i: (i, 0)),
        out_shape=jax.ShapeDtypeStruct((B, H), jnp.float32),
    )(emb_seq, W_ihT, W_hhT, bias2d)


def _fc_body(h_ref, w_ref, b_ref, out_ref):
    out_ref[...] = (
        lax.dot_general(
            h_ref[...], w_ref[...],
            dimension_numbers=(((1,), (1,)), ((), ())),
            preferred_element_type=jnp.float32,
        )
        + b_ref[...]
    )


def _fc(h, W_fc, b_fc2d):
    B, H = h.shape
    V = W_fc.shape[0]
    VB = 4096
    nv = pl.cdiv(V, VB)
    return pl.pallas_call(
        _fc_body,
        grid=(nv,),
        in_specs=[
            pl.BlockSpec((B, H), lambda i: (0, 0)),
            pl.BlockSpec((VB, H), lambda i: (i, 0)),
            pl.BlockSpec((1, VB), lambda i: (0, i)),
        ],
        out_specs=pl.BlockSpec((B, VB), lambda i: (0, i)),
        out_shape=jax.ShapeDtypeStruct((B, V), jnp.float32),
    )(h, W_fc, b_fc2d)


def kernel(x, emb, W_ih, W_hh, b_ih, b_hh, W_fc, b_fc):
    B, T = x.shape
    V, E = emb.shape
    H = W_hh.shape[1]

    CH = 128
    idx_flat = x.astype(jnp.int32).reshape(-1)

    embedded = _sc_gather(emb, idx_flat, E, CH).reshape(B, T, E)

    h_last = _lstm(
        embedded,
        W_ih.T,
        W_hh.T,
        (b_ih + b_hh).reshape(1, 4 * H),
    )

    return _fc(h_last.astype(jnp.bfloat16), W_fc.astype(jnp.bfloat16),
               b_fc.reshape(1, V))
